# baseline (speedup 1.0000x reference)
import functools

import jax
import jax.numpy as jnp
from jax import lax
from jax.experimental import pallas as pl
from jax.experimental.pallas import tpu as pltpu
from jax.experimental.pallas import tpu_sc as plsc

D = 64
B, T, N = 8, 64, 1024

NC, NS = 2, 16
NW = NC * NS
NCHUNK = N // NW
NJ = D // 16


def _sc_base_kernel(regions_hbm, eids_hbm, bsp_hbm, slot_hbm, remb_hbm,
                    eemb_hbm, base_hbm, idx_v, reg_v, slot_v, out_v,
                    eids_v, eid_rows_v, bsp_v, cb_v, sem):
    c = lax.axis_index("c")
    s = lax.axis_index("s")
    wid = s * NC + c
    nbase = wid * NCHUNK

    pltpu.sync_copy(slot_hbm.at[pl.ds(nbase, NCHUNK)], slot_v)
    pltpu.sync_copy(eids_hbm, eids_v)
    pltpu.sync_copy(bsp_hbm, bsp_v)
    pltpu.async_copy(eemb_hbm.at[eids_v], eid_rows_v, sem).wait()

    for b in range(B):
        pltpu.sync_copy(regions_hbm.at[pl.ds(b * N + nbase, NCHUNK)], idx_v)
        pltpu.async_copy(remb_hbm.at[idx_v], reg_v, sem).wait()
        for j in range(NJ):
            sl = pl.ds(16 * j, 16)
            cb_v[sl] = eid_rows_v[b, sl] + bsp_v[sl]

        def body(n, carry):
            for j in range(NJ):
                sl = pl.ds(16 * j, 16)
                out_v[n, sl] = slot_v[n, sl] + reg_v[n, sl] + cb_v[sl]
            return carry

        lax.fori_loop(0, NCHUNK, body, 0)
        pltpu.sync_copy(out_v, base_hbm.at[pl.ds(b * N + nbase, NCHUNK)])


@functools.lru_cache(maxsize=1)
def _sc_base():
    return pl.kernel(
        _sc_base_kernel,
        out_type=jax.ShapeDtypeStruct((B * N, D), jnp.float32),
        mesh=plsc.VectorSubcoreMesh(core_axis_name="c", subcore_axis_name="s",
                                    num_cores=NC, num_subcores=NS),
        scratch_types=[
            pltpu.VMEM((NCHUNK,), jnp.int32),
            pltpu.VMEM((NCHUNK, D), jnp.float32),
            pltpu.VMEM((NCHUNK, D), jnp.float32),
            pltpu.VMEM((NCHUNK, D), jnp.float32),
            pltpu.VMEM((B,), jnp.int32),
            pltpu.VMEM((B, D), jnp.float32),
            pltpu.VMEM((D,), jnp.float32),
            pltpu.VMEM((D,), jnp.float32),
            pltpu.SemaphoreType.DMA,
        ],
        compiler_params=pltpu.CompilerParams(use_tc_tiling_on_sc=False),
    )


TT = 8
ND = N * D
CN = 16
CD = CN * D


def _tc_body(s_ref, ew_ref, base_ref, o_ref):
    s = s_ref[0]
    ew = ew_ref[...]
    for c in range(N // CN):
        sl = slice(c * CD, (c + 1) * CD)
        prod = jax.lax.dot(s[:, c * CN:(c + 1) * CN], ew,
                           precision=jax.lax.Precision.DEFAULT,
                           preferred_element_type=jnp.float32)
        o_ref[0, :, sl] = prod + base_ref[0, :, sl]


def _tc_broadcast(spikes, ew, base_flat):
    out = pl.pallas_call(
        _tc_body,
        grid=(B, T // TT),
        in_specs=[
            pl.BlockSpec((1, TT, N), lambda i, j: (i, j, 0)),
            pl.BlockSpec((CN, CD), lambda i, j: (0, 0)),
            pl.BlockSpec((1, 1, ND), lambda i, j: (i, 0, 0)),
        ],
        out_specs=pl.BlockSpec((1, TT, ND), lambda i, j: (i, j, 0)),
        out_shape=jax.ShapeDtypeStruct((B, T, ND), jnp.float32),
    )(spikes, ew, base_flat)
    return out.reshape(B, T, N, D)


def kernel(spikes, neuron_regions, eids, w_spike, b_spike, neuron_slot,
           region_emb, eid_emb):
    regions_flat = neuron_regions.astype(jnp.int32).reshape(B * N)
    base = _sc_base()(regions_flat, eids.astype(jnp.int32), b_spike,
                      neuron_slot, region_emb, eid_emb)
    rows = jnp.arange(CN, dtype=jnp.int32)[:, None]
    cols = jnp.arange(CD, dtype=jnp.int32)[None, :]
    ew = jnp.where(rows == cols // D, 1.0, 0.0) * w_spike[cols % D, 0]
    return _tc_broadcast(spikes, ew, base.reshape(B, 1, ND))


# speedup vs baseline: 1.0154x; 1.0154x over previous
import functools

import jax
import jax.numpy as jnp
from jax import lax
from jax.experimental import pallas as pl
from jax.experimental.pallas import tpu as pltpu
from jax.experimental.pallas import tpu_sc as plsc

D = 64
B, T, N = 8, 64, 1024

NC, NS = 2, 16
NW = NC * NS
WB = NW // B
NCHUNK = N // WB
NJ = D // 16


def _sc_base_kernel(regions_hbm, eids_hbm, bsp_hbm, slot_hbm, remb_hbm,
                    eemb_hbm, base_hbm, idx_v, reg_v, slot_v, out_v,
                    eids_v, eid_rows_v, bsp_v, cb_v, sem, sem2):
    c = lax.axis_index("c")
    s = lax.axis_index("s")
    wid = s * NC + c
    b = wid // WB
    nbase = (wid % WB) * NCHUNK

    slot_cp = pltpu.async_copy(slot_hbm.at[pl.ds(nbase, NCHUNK)], slot_v, sem2)
    pltpu.sync_copy(eids_hbm, eids_v)
    pltpu.sync_copy(bsp_hbm, bsp_v)
    pltpu.async_copy(eemb_hbm.at[eids_v], eid_rows_v, sem).wait()
    pltpu.sync_copy(regions_hbm.at[pl.ds(b * N + nbase, NCHUNK)], idx_v)
    pltpu.async_copy(remb_hbm.at[idx_v], reg_v, sem).wait()
    slot_cp.wait()

    for j in range(NJ):
        sl = pl.ds(16 * j, 16)
        cb_v[sl] = eid_rows_v[b, sl] + bsp_v[sl]

    def body(n, carry):
        for j in range(NJ):
            sl = pl.ds(16 * j, 16)
            out_v[n, sl] = slot_v[n, sl] + reg_v[n, sl] + cb_v[sl]
        return carry

    lax.fori_loop(0, NCHUNK, body, 0)
    pltpu.sync_copy(out_v, base_hbm.at[pl.ds(b * N + nbase, NCHUNK)])


@functools.lru_cache(maxsize=1)
def _sc_base():
    return pl.kernel(
        _sc_base_kernel,
        out_type=jax.ShapeDtypeStruct((B * N, D), jnp.float32),
        mesh=plsc.VectorSubcoreMesh(core_axis_name="c", subcore_axis_name="s",
                                    num_cores=NC, num_subcores=NS),
        scratch_types=[
            pltpu.VMEM((NCHUNK,), jnp.int32),
            pltpu.VMEM((NCHUNK, D), jnp.float32),
            pltpu.VMEM((NCHUNK, D), jnp.float32),
            pltpu.VMEM((NCHUNK, D), jnp.float32),
            pltpu.VMEM((B,), jnp.int32),
            pltpu.VMEM((B, D), jnp.float32),
            pltpu.VMEM((D,), jnp.float32),
            pltpu.VMEM((D,), jnp.float32),
            pltpu.SemaphoreType.DMA,
            pltpu.SemaphoreType.DMA,
        ],
        compiler_params=pltpu.CompilerParams(use_tc_tiling_on_sc=False),
    )


TT = 8
ND = N * D
CN = 16
CD = CN * D


def _tc_body(s_ref, ew_ref, base_ref, o_ref):
    s = s_ref[0]
    ew = ew_ref[...]
    for c in range(N // CN):
        sl = slice(c * CD, (c + 1) * CD)
        prod = jax.lax.dot(s[:, c * CN:(c + 1) * CN], ew,
                           precision=jax.lax.Precision.DEFAULT,
                           preferred_element_type=jnp.float32)
        o_ref[0, :, sl] = prod + base_ref[0, :, sl]


def _tc_broadcast(spikes, ew, base_flat):
    out = pl.pallas_call(
        _tc_body,
        grid=(B, T // TT),
        in_specs=[
            pl.BlockSpec((1, TT, N), lambda i, j: (i, j, 0)),
            pl.BlockSpec((CN, CD), lambda i, j: (0, 0)),
            pl.BlockSpec((1, 1, ND), lambda i, j: (i, 0, 0)),
        ],
        out_specs=pl.BlockSpec((1, TT, ND), lambda i, j: (i, j, 0)),
        out_shape=jax.ShapeDtypeStruct((B, T, ND), jnp.float32),
    )(spikes, ew, base_flat)
    return out.reshape(B, T, N, D)


def kernel(spikes, neuron_regions, eids, w_spike, b_spike, neuron_slot,
           region_emb, eid_emb):
    regions_flat = neuron_regions.astype(jnp.int32).reshape(B * N)
    base = _sc_base()(regions_flat, eids.astype(jnp.int32), b_spike,
                      neuron_slot, region_emb, eid_emb)
    rows = jnp.arange(CN, dtype=jnp.int32)[:, None]
    cols = jnp.arange(CD, dtype=jnp.int32)[None, :]
    ew = jnp.where(rows == cols // D, 1.0, 0.0) * w_spike[cols % D, 0]
    return _tc_broadcast(spikes, ew, base.reshape(B, 1, ND))


# speedup vs baseline: 3.1648x; 3.1169x over previous
import functools

import jax
import jax.numpy as jnp
from jax import lax
from jax.experimental import pallas as pl
from jax.experimental.pallas import tpu as pltpu
from jax.experimental.pallas import tpu_sc as plsc

D = 64
B, T, N = 8, 64, 1024

NC, NS = 2, 16
NW = NC * NS
WB = NW // B
NCHUNK = N // WB
NJ = D // 16


def _sc_base_kernel(regions_hbm, eids_hbm, bsp_hbm, slot_hbm, remb_hbm,
                    eemb_hbm, base_hbm, idx_v, reg_v, slot_v, out_v,
                    eids_v, eid_rows_v, bsp_v, cb_v, sem, sem2):
    c = lax.axis_index("c")
    s = lax.axis_index("s")
    wid = s * NC + c
    b = wid // WB
    nbase = (wid % WB) * NCHUNK

    slot_cp = pltpu.async_copy(slot_hbm.at[pl.ds(nbase, NCHUNK)], slot_v, sem2)
    pltpu.sync_copy(eids_hbm, eids_v)
    pltpu.sync_copy(bsp_hbm, bsp_v)
    pltpu.async_copy(eemb_hbm.at[eids_v], eid_rows_v, sem).wait()
    pltpu.sync_copy(regions_hbm.at[pl.ds(b * N + nbase, NCHUNK)], idx_v)
    pltpu.async_copy(remb_hbm.at[idx_v], reg_v, sem).wait()
    slot_cp.wait()

    for j in range(NJ):
        sl = pl.ds(16 * j, 16)
        cb_v[sl] = eid_rows_v[b, sl] + bsp_v[sl]

    def body(n, carry):
        for j in range(NJ):
            sl = pl.ds(16 * j, 16)
            out_v[n, sl] = slot_v[n, sl] + reg_v[n, sl] + cb_v[sl]
        return carry

    lax.fori_loop(0, NCHUNK, body, 0)
    pltpu.sync_copy(out_v, base_hbm.at[pl.ds(b * N + nbase, NCHUNK)])


@functools.lru_cache(maxsize=1)
def _sc_base():
    return pl.kernel(
        _sc_base_kernel,
        out_type=jax.ShapeDtypeStruct((B * N, D), jnp.float32),
        mesh=plsc.VectorSubcoreMesh(core_axis_name="c", subcore_axis_name="s",
                                    num_cores=NC, num_subcores=NS),
        scratch_types=[
            pltpu.VMEM((NCHUNK,), jnp.int32),
            pltpu.VMEM((NCHUNK, D), jnp.float32),
            pltpu.VMEM((NCHUNK, D), jnp.float32),
            pltpu.VMEM((NCHUNK, D), jnp.float32),
            pltpu.VMEM((B,), jnp.int32),
            pltpu.VMEM((B, D), jnp.float32),
            pltpu.VMEM((D,), jnp.float32),
            pltpu.VMEM((D,), jnp.float32),
            pltpu.SemaphoreType.DMA,
            pltpu.SemaphoreType.DMA,
        ],
        compiler_params=pltpu.CompilerParams(use_tc_tiling_on_sc=False),
    )


TT = 8


def _tc_body(s_ref, w_ref, base_ref, o_ref, basep_s):
    @pl.when(pl.program_id(1) == 0)
    def _():
        basep_s[...] = jnp.transpose(base_ref[0], (1, 0))

    s = s_ref[0]
    w = w_ref[...]
    o_ref[0] = (s[:, None, :] * w[None, :, :] + basep_s[...][None, :, :])


def _tc_broadcast(spikes, w_spike, base):
    outp = pl.pallas_call(
        _tc_body,
        grid=(B, T // TT),
        in_specs=[
            pl.BlockSpec((1, TT, N), lambda i, j: (i, j, 0)),
            pl.BlockSpec((D, 1), lambda i, j: (0, 0)),
            pl.BlockSpec((1, N, D), lambda i, j: (i, 0, 0)),
        ],
        out_specs=pl.BlockSpec((1, TT, D, N), lambda i, j: (i, j, 0, 0)),
        out_shape=jax.ShapeDtypeStruct((B, T, D, N), jnp.float32),
        scratch_shapes=[pltpu.VMEM((D, N), jnp.float32)],
    )(spikes, w_spike, base)
    return jnp.swapaxes(outp, 2, 3)


def kernel(spikes, neuron_regions, eids, w_spike, b_spike, neuron_slot,
           region_emb, eid_emb):
    regions_flat = neuron_regions.astype(jnp.int32).reshape(B * N)
    base = _sc_base()(regions_flat, eids.astype(jnp.int32), b_spike,
                      neuron_slot, region_emb, eid_emb)
    return _tc_broadcast(spikes, w_spike, base.reshape(B, N, D))


# speedup vs baseline: 3.7377x; 1.1810x over previous
import functools

import jax
import jax.numpy as jnp
from jax import lax
from jax.experimental import pallas as pl
from jax.experimental.pallas import tpu as pltpu
from jax.experimental.pallas import tpu_sc as plsc

D = 64
B, T, N = 8, 64, 1024

NC, NS = 2, 16
NW = NC * NS
WB = NW // B
NCHUNK = N // WB
NJ = D // 16


def _sc_gather_kernel(regions_hbm, eids_hbm, bsp_hbm, remb_hbm, eemb_hbm,
                      reg_out_hbm, cb_out_hbm, idx_v, reg_v, eids_v,
                      eid_rows_v, bsp_v, sem):
    c = lax.axis_index("c")
    s = lax.axis_index("s")
    wid = s * NC + c
    b = wid // WB
    nbase = (wid % WB) * NCHUNK

    pltpu.sync_copy(regions_hbm.at[pl.ds(b * N + nbase, NCHUNK)], idx_v)
    pltpu.async_copy(remb_hbm.at[idx_v], reg_v, sem).wait()
    pltpu.sync_copy(reg_v, reg_out_hbm.at[pl.ds(b * N + nbase, NCHUNK)])

    @pl.when(wid == 0)
    def _():
        pltpu.sync_copy(eids_hbm, eids_v)
        pltpu.sync_copy(bsp_hbm, bsp_v)
        pltpu.async_copy(eemb_hbm.at[eids_v], eid_rows_v, sem).wait()
        for bb in range(B):
            for j in range(NJ):
                sl = pl.ds(16 * j, 16)
                eid_rows_v[bb, sl] = eid_rows_v[bb, sl] + bsp_v[sl]
        pltpu.sync_copy(eid_rows_v, cb_out_hbm)


@functools.lru_cache(maxsize=1)
def _sc_gather():
    return pl.kernel(
        _sc_gather_kernel,
        out_type=(jax.ShapeDtypeStruct((B * N, D), jnp.float32),
                  jax.ShapeDtypeStruct((B, D), jnp.float32)),
        mesh=plsc.VectorSubcoreMesh(core_axis_name="c", subcore_axis_name="s",
                                    num_cores=NC, num_subcores=NS),
        scratch_types=[
            pltpu.VMEM((NCHUNK,), jnp.int32),
            pltpu.VMEM((NCHUNK, D), jnp.float32),
            pltpu.VMEM((B,), jnp.int32),
            pltpu.VMEM((B, D), jnp.float32),
            pltpu.VMEM((D,), jnp.float32),
            pltpu.SemaphoreType.DMA,
        ],
        compiler_params=pltpu.CompilerParams(use_tc_tiling_on_sc=False),
    )


TT = 16


def _tc_body(s_ref, w_ref, slot_ref, reg_ref, cbt_ref, o_ref, baset_s):
    @pl.when(pl.program_id(1) == 0)
    def _():
        cbt = cbt_ref[...]
        bsel = jax.lax.broadcasted_iota(jnp.int32, (D, B), 1) == pl.program_id(0)
        cb_col = jnp.sum(jnp.where(bsel, cbt, 0.0), axis=1, keepdims=True)
        baset_s[...] = (jnp.transpose(reg_ref[0], (1, 0))
                        + jnp.transpose(slot_ref[...], (1, 0))
                        + cb_col)

    s = s_ref[0]
    w = w_ref[...]
    o_ref[0] = s[:, None, :] * w[None, :, :] + baset_s[...][None, :, :]


def _tc_broadcast(spikes, w_spike, slot, reg, cbt):
    outp = pl.pallas_call(
        _tc_body,
        grid=(B, T // TT),
        in_specs=[
            pl.BlockSpec((1, TT, N), lambda i, j: (i, j, 0)),
            pl.BlockSpec((D, 1), lambda i, j: (0, 0)),
            pl.BlockSpec((N, D), lambda i, j: (0, 0)),
            pl.BlockSpec((1, N, D), lambda i, j: (i, 0, 0)),
            pl.BlockSpec((D, B), lambda i, j: (0, 0)),
        ],
        out_specs=pl.BlockSpec((1, TT, D, N), lambda i, j: (i, j, 0, 0)),
        out_shape=jax.ShapeDtypeStruct((B, T, D, N), jnp.float32),
        scratch_shapes=[pltpu.VMEM((D, N), jnp.float32)],
    )(spikes, w_spike, slot, reg, cbt)
    return jnp.swapaxes(outp, 2, 3)


def kernel(spikes, neuron_regions, eids, w_spike, b_spike, neuron_slot,
           region_emb, eid_emb):
    regions_flat = neuron_regions.astype(jnp.int32).reshape(B * N)
    reg, cb = _sc_gather()(regions_flat, eids.astype(jnp.int32), b_spike,
                           region_emb, eid_emb)
    return _tc_broadcast(spikes, w_spike, neuron_slot,
                         reg.reshape(B, N, D), cb.T)


# speedup vs baseline: 4.1033x; 1.0978x over previous
import functools

import jax
import jax.numpy as jnp
from jax import lax
from jax.experimental import pallas as pl
from jax.experimental.pallas import tpu as pltpu
from jax.experimental.pallas import tpu_sc as plsc

D = 64
B, T, N = 8, 64, 1024

NC, NS = 2, 16
NW = NC * NS
WB = NW // B
NCHUNK = N // WB
NJ = D // 16


def _sc_gather_kernel(regions_hbm, eids_hbm, bsp_hbm, remb_hbm, eemb_hbm,
                      reg_out_hbm, cb_out_hbm, idx_v, reg_v, eids_v,
                      eid_rows_v, bsp_v, sem):
    c = lax.axis_index("c")
    s = lax.axis_index("s")
    wid = s * NC + c
    b = wid // WB
    nbase = (wid % WB) * NCHUNK

    pltpu.sync_copy(regions_hbm.at[pl.ds(b * N + nbase, NCHUNK)], idx_v)
    pltpu.async_copy(remb_hbm.at[idx_v], reg_v, sem).wait()
    pltpu.sync_copy(reg_v, reg_out_hbm.at[pl.ds(b * N + nbase, NCHUNK)])

    @pl.when(wid == 0)
    def _():
        pltpu.sync_copy(eids_hbm, eids_v)
        pltpu.sync_copy(bsp_hbm, bsp_v)
        pltpu.async_copy(eemb_hbm.at[eids_v], eid_rows_v, sem).wait()
        for bb in range(B):
            for j in range(NJ):
                sl = pl.ds(16 * j, 16)
                eid_rows_v[bb, sl] = eid_rows_v[bb, sl] + bsp_v[sl]
        pltpu.sync_copy(eid_rows_v, cb_out_hbm)


@functools.lru_cache(maxsize=1)
def _sc_gather():
    return pl.kernel(
        _sc_gather_kernel,
        out_type=(jax.ShapeDtypeStruct((B * N, D), jnp.float32),
                  jax.ShapeDtypeStruct((B, D), jnp.float32)),
        mesh=plsc.VectorSubcoreMesh(core_axis_name="c", subcore_axis_name="s",
                                    num_cores=NC, num_subcores=NS),
        scratch_types=[
            pltpu.VMEM((NCHUNK,), jnp.int32),
            pltpu.VMEM((NCHUNK, D), jnp.float32),
            pltpu.VMEM((B,), jnp.int32),
            pltpu.VMEM((B, D), jnp.float32),
            pltpu.VMEM((D,), jnp.float32),
            pltpu.SemaphoreType.DMA,
        ],
        compiler_params=pltpu.CompilerParams(use_tc_tiling_on_sc=False),
    )


TT = 32


def _tc_body(s_ref, w_ref, slot_ref, reg_ref, cbt_ref, o_ref, baset_s):
    @pl.when(pl.program_id(1) == 0)
    def _():
        cbt = cbt_ref[...]
        bsel = jax.lax.broadcasted_iota(jnp.int32, (D, B), 1) == pl.program_id(0)
        cb_col = jnp.sum(jnp.where(bsel, cbt, 0.0), axis=1, keepdims=True)
        baset_s[...] = (jnp.transpose(reg_ref[0], (1, 0))
                        + jnp.transpose(slot_ref[...], (1, 0))
                        + cb_col)

    s = s_ref[0]
    w = w_ref[...]
    o_ref[0] = s[:, None, :] * w[None, :, :] + baset_s[...][None, :, :]


def _tc_broadcast(spikes, w_spike, slot, reg, cbt):
    outp = pl.pallas_call(
        _tc_body,
        grid=(B, T // TT),
        in_specs=[
            pl.BlockSpec((1, TT, N), lambda i, j: (i, j, 0)),
            pl.BlockSpec((D, 1), lambda i, j: (0, 0)),
            pl.BlockSpec((N, D), lambda i, j: (0, 0)),
            pl.BlockSpec((1, N, D), lambda i, j: (i, 0, 0)),
            pl.BlockSpec((D, B), lambda i, j: (0, 0)),
        ],
        out_specs=pl.BlockSpec((1, TT, D, N), lambda i, j: (i, j, 0, 0)),
        out_shape=jax.ShapeDtypeStruct((B, T, D, N), jnp.float32),
        scratch_shapes=[pltpu.VMEM((D, N), jnp.float32)],
    )(spikes, w_spike, slot, reg, cbt)
    return jnp.swapaxes(outp, 2, 3)


def kernel(spikes, neuron_regions, eids, w_spike, b_spike, neuron_slot,
           region_emb, eid_emb):
    regions_flat = neuron_regions.astype(jnp.int32).reshape(B * N)
    reg, cb = _sc_gather()(regions_flat, eids.astype(jnp.int32), b_spike,
                           region_emb, eid_emb)
    return _tc_broadcast(spikes, w_spike, neuron_slot,
                         reg.reshape(B, N, D), cb.T)


# speedup vs baseline: 4.1382x; 1.0085x over previous
import functools

import jax
import jax.numpy as jnp
from jax import lax
from jax.experimental import pallas as pl
from jax.experimental.pallas import tpu as pltpu
from jax.experimental.pallas import tpu_sc as plsc

D = 64
B, T, N = 8, 64, 1024

NC, NS = 2, 16
NW = NC * NS
WB = NW // B
NCHUNK = N // WB
NJ = D // 16


def _sc_gather_kernel(regions_hbm, eids_hbm, bsp_hbm, remb_hbm, eemb_hbm,
                      reg_out_hbm, cb_out_hbm, idx_v, reg_v, eids_v,
                      eid_rows_v, bsp_v, sem):
    c = lax.axis_index("c")
    s = lax.axis_index("s")
    wid = s * NC + c
    b = wid // WB
    nbase = (wid % WB) * NCHUNK

    pltpu.sync_copy(regions_hbm.at[pl.ds(b * N + nbase, NCHUNK)], idx_v)
    pltpu.async_copy(remb_hbm.at[idx_v], reg_v, sem).wait()
    pltpu.sync_copy(reg_v, reg_out_hbm.at[pl.ds(b * N + nbase, NCHUNK)])

    @pl.when(wid == 0)
    def _():
        pltpu.sync_copy(eids_hbm, eids_v)
        pltpu.sync_copy(bsp_hbm, bsp_v)
        pltpu.async_copy(eemb_hbm.at[eids_v], eid_rows_v, sem).wait()
        for bb in range(B):
            for j in range(NJ):
                sl = pl.ds(16 * j, 16)
                eid_rows_v[bb, sl] = eid_rows_v[bb, sl] + bsp_v[sl]
        pltpu.sync_copy(eid_rows_v, cb_out_hbm)


@functools.lru_cache(maxsize=1)
def _sc_gather():
    return pl.kernel(
        _sc_gather_kernel,
        out_type=(jax.ShapeDtypeStruct((B * N, D), jnp.float32),
                  jax.ShapeDtypeStruct((B, D), jnp.float32)),
        mesh=plsc.VectorSubcoreMesh(core_axis_name="c", subcore_axis_name="s",
                                    num_cores=NC, num_subcores=NS),
        scratch_types=[
            pltpu.VMEM((NCHUNK,), jnp.int32),
            pltpu.VMEM((NCHUNK, D), jnp.float32),
            pltpu.VMEM((B,), jnp.int32),
            pltpu.VMEM((B, D), jnp.float32),
            pltpu.VMEM((D,), jnp.float32),
            pltpu.SemaphoreType.DMA,
        ],
        compiler_params=pltpu.CompilerParams(use_tc_tiling_on_sc=False),
    )


TT = 64


def _tc_body(s_ref, w_ref, slot_ref, reg_ref, cbt_ref, o_ref, baset_s):
    @pl.when(pl.program_id(1) == 0)
    def _():
        cbt = cbt_ref[...]
        bsel = jax.lax.broadcasted_iota(jnp.int32, (D, B), 1) == pl.program_id(0)
        cb_col = jnp.sum(jnp.where(bsel, cbt, 0.0), axis=1, keepdims=True)
        baset_s[...] = (jnp.transpose(reg_ref[0], (1, 0))
                        + jnp.transpose(slot_ref[...], (1, 0))
                        + cb_col)

    s = s_ref[0]
    w = w_ref[...]
    o_ref[0] = s[:, None, :] * w[None, :, :] + baset_s[...][None, :, :]


def _tc_broadcast(spikes, w_spike, slot, reg, cbt):
    outp = pl.pallas_call(
        _tc_body,
        grid=(B, T // TT),
        in_specs=[
            pl.BlockSpec((1, TT, N), lambda i, j: (i, j, 0)),
            pl.BlockSpec((D, 1), lambda i, j: (0, 0)),
            pl.BlockSpec((N, D), lambda i, j: (0, 0)),
            pl.BlockSpec((1, N, D), lambda i, j: (i, 0, 0)),
            pl.BlockSpec((D, B), lambda i, j: (0, 0)),
        ],
        out_specs=pl.BlockSpec((1, TT, D, N), lambda i, j: (i, j, 0, 0)),
        out_shape=jax.ShapeDtypeStruct((B, T, D, N), jnp.float32),
        scratch_shapes=[pltpu.VMEM((D, N), jnp.float32)],
    )(spikes, w_spike, slot, reg, cbt)
    return jnp.swapaxes(outp, 2, 3)


def kernel(spikes, neuron_regions, eids, w_spike, b_spike, neuron_slot,
           region_emb, eid_emb):
    regions_flat = neuron_regions.astype(jnp.int32).reshape(B * N)
    reg, cb = _sc_gather()(regions_flat, eids.astype(jnp.int32), b_spike,
                           region_emb, eid_emb)
    return _tc_broadcast(spikes, w_spike, neuron_slot,
                         reg.reshape(B, N, D), cb.T)
